# baseline (speedup 1.0000x reference)
import functools

import jax
import jax.numpy as jnp
from jax import lax
from jax.experimental import pallas as pl
from jax.experimental.pallas import tpu as pltpu
from jax.experimental.pallas import tpu_sc as plsc

VOCAB = 1000000
D = 64
B = 4096 * 200
NC, NS = 2, 16
NW = NC * NS
BPW = B // NW
CHUNK = 128
NCHUNK = BPW // CHUNK
NBUF = 2


def _sc_gather(table, idx_flat):
  mesh = plsc.VectorSubcoreMesh(
      core_axis_name="c", subcore_axis_name="s", num_cores=NC, num_subcores=NS
  )

  @functools.partial(
      pl.kernel,
      out_type=jax.ShapeDtypeStruct((B, D), jnp.float32),
      mesh=mesh,
      scratch_types=[
          pltpu.VMEM((NCHUNK, CHUNK), jnp.int32),
          pltpu.VMEM((NBUF, CHUNK, D), jnp.float32),
          pltpu.SemaphoreType.DMA((NBUF,)),
      ],
      compiler_params=pltpu.CompilerParams(use_tc_tiling_on_sc=False),
  )
  def k(table_hbm, idx_hbm, out_hbm, idx_v, rows_v, sem):
    wid = lax.axis_index("s") * NC + lax.axis_index("c")
    base = wid * BPW
    pltpu.sync_copy(idx_hbm.at[pl.ds(wid * NCHUNK, NCHUNK)], idx_v)

    def gather(j, buf):
      return pltpu.async_copy(
          table_hbm.at[idx_v.at[j]], rows_v.at[buf], sem.at[buf]
      )

    gather(0, 0)

    def body(j, _):
      buf = lax.rem(j, NBUF)

      @pl.when(j + 1 < NCHUNK)
      def _prefetch():
        gather(j + 1, lax.rem(j + 1, NBUF))

      pltpu.make_async_copy(
          table_hbm.at[idx_v.at[j]], rows_v.at[buf], sem.at[buf]
      ).wait()
      pltpu.sync_copy(
          rows_v.at[buf], out_hbm.at[pl.ds(base + j * CHUNK, CHUNK)]
      )
      return _

    lax.fori_loop(0, NCHUNK, body, None)

  return k(table, idx_flat.reshape(B // CHUNK, CHUNK))


def kernel(x, table):
  out = _sc_gather(table, x.reshape(-1).astype(jnp.int32))
  return out.reshape(x.shape[0], x.shape[1], D)


# speedup vs baseline: 1.0190x; 1.0190x over previous
import functools

import jax
import jax.numpy as jnp
from jax import lax
from jax.experimental import pallas as pl
from jax.experimental.pallas import tpu as pltpu
from jax.experimental.pallas import tpu_sc as plsc

VOCAB = 1000000
D = 64
B = 4096 * 200
NC, NS = 2, 16
NW = NC * NS
BPW = B // NW
CHUNK = 128
NCHUNK = BPW // CHUNK
NBUF = 6
AHEAD = 4
SLACK = NBUF - AHEAD


def _sc_gather(table, idx_flat):
  mesh = plsc.VectorSubcoreMesh(
      core_axis_name="c", subcore_axis_name="s", num_cores=NC, num_subcores=NS
  )

  @functools.partial(
      pl.kernel,
      out_type=jax.ShapeDtypeStruct((B, D), jnp.float32),
      mesh=mesh,
      scratch_types=[
          pltpu.VMEM((NCHUNK, CHUNK), jnp.int32),
          pltpu.VMEM((NBUF, CHUNK, D), jnp.float32),
          pltpu.SemaphoreType.DMA((NBUF,)),
          pltpu.SemaphoreType.DMA((NBUF,)),
      ],
      compiler_params=pltpu.CompilerParams(use_tc_tiling_on_sc=False),
  )
  def k(table_hbm, idx_hbm, out_hbm, idx_v, rows_v, gsem, wsem):
    wid = lax.axis_index("s") * NC + lax.axis_index("c")
    base = wid * BPW
    pltpu.sync_copy(idx_hbm.at[pl.ds(wid * NCHUNK, NCHUNK)], idx_v)

    def gather(j, buf):
      pltpu.async_copy(table_hbm.at[idx_v.at[j]], rows_v.at[buf], gsem.at[buf])

    def gather_wait(j, buf):
      pltpu.make_async_copy(
          table_hbm.at[idx_v.at[j]], rows_v.at[buf], gsem.at[buf]
      ).wait()

    def write(j, buf):
      pltpu.async_copy(
          rows_v.at[buf], out_hbm.at[pl.ds(base + j * CHUNK, CHUNK)],
          wsem.at[buf],
      )

    def write_wait(j, buf):
      pltpu.make_async_copy(
          rows_v.at[buf], out_hbm.at[pl.ds(base + j * CHUNK, CHUNK)],
          wsem.at[buf],
      ).wait()

    for b in range(AHEAD):
      gather(b, b)

    def body(j, _):
      g = j + AHEAD
      buf_g = lax.rem(g, NBUF)

      @pl.when(g < NCHUNK)
      def _refill():
        @pl.when(j >= SLACK)
        def _drain():
          write_wait(j - SLACK, buf_g)

        gather(g, buf_g)

      buf = lax.rem(j, NBUF)
      gather_wait(j, buf)
      write(j, buf)
      return _

    lax.fori_loop(0, NCHUNK, body, None)

    for t in range(NBUF):
      j = NCHUNK - NBUF + t
      write_wait(j, j % NBUF)

  return k(table, idx_flat.reshape(B // CHUNK, CHUNK))


def kernel(x, table):
  out = _sc_gather(table, x.reshape(-1).astype(jnp.int32))
  return out.reshape(x.shape[0], x.shape[1], D)


# speedup vs baseline: 1.0212x; 1.0021x over previous
import functools

import jax
import jax.numpy as jnp
from jax import lax
from jax.experimental import pallas as pl
from jax.experimental.pallas import tpu as pltpu
from jax.experimental.pallas import tpu_sc as plsc

VOCAB = 1000000
D = 64
R, C = 4096, 200
NC, NS = 2, 16
NW = NC * NS
RPW = R // NW
CA, CB = 104, 96
NCHUNK = RPW * 2
NBUF = 6
AHEAD = 4
SLACK = NBUF - AHEAD


def _sc_gather(x, table):
  mesh = plsc.VectorSubcoreMesh(
      core_axis_name="c", subcore_axis_name="s", num_cores=NC, num_subcores=NS
  )

  @functools.partial(
      pl.kernel,
      out_type=jax.ShapeDtypeStruct((R, C, D), jnp.float32),
      mesh=mesh,
      scratch_types=[
          pltpu.VMEM((RPW, C), jnp.int32),
          pltpu.VMEM((NBUF, CA, D), jnp.float32),
          pltpu.SemaphoreType.DMA((NBUF,)),
          pltpu.SemaphoreType.DMA((NBUF,)),
      ],
      compiler_params=pltpu.CompilerParams(use_tc_tiling_on_sc=False),
  )
  def k(table_hbm, x_hbm, out_hbm, idx_v, rows_v, gsem, wsem):
    wid = lax.axis_index("s") * NC + lax.axis_index("c")
    row0 = wid * RPW
    pltpu.sync_copy(x_hbm.at[pl.ds(row0, RPW)], idx_v)

    def copies(j, buf):
      r = lax.div(j, 2)
      ga = pltpu.make_async_copy(
          table_hbm.at[idx_v.at[r, pl.ds(0, CA)]],
          rows_v.at[buf],
          gsem.at[buf],
      )
      gb = pltpu.make_async_copy(
          table_hbm.at[idx_v.at[r, pl.ds(CA, CB)]],
          rows_v.at[buf, pl.ds(0, CB)],
          gsem.at[buf],
      )
      wa = pltpu.make_async_copy(
          rows_v.at[buf],
          out_hbm.at[row0 + r, pl.ds(0, CA)],
          wsem.at[buf],
      )
      wb = pltpu.make_async_copy(
          rows_v.at[buf, pl.ds(0, CB)],
          out_hbm.at[row0 + r, pl.ds(CA, CB)],
          wsem.at[buf],
      )
      return ga, gb, wa, wb

    def gather(j, buf):
      ga, gb, _, _ = copies(j, buf)
      even = lax.rem(j, 2) == 0

      @pl.when(even)
      def _():
        ga.start()

      @pl.when(jnp.logical_not(even))
      def _():
        gb.start()

    def gather_wait(j, buf):
      ga, gb, _, _ = copies(j, buf)
      even = lax.rem(j, 2) == 0

      @pl.when(even)
      def _():
        ga.wait()

      @pl.when(jnp.logical_not(even))
      def _():
        gb.wait()

    def write(j, buf):
      _, _, wa, wb = copies(j, buf)
      even = lax.rem(j, 2) == 0

      @pl.when(even)
      def _():
        wa.start()

      @pl.when(jnp.logical_not(even))
      def _():
        wb.start()

    def write_wait(j, buf):
      _, _, wa, wb = copies(j, buf)
      even = lax.rem(j, 2) == 0

      @pl.when(even)
      def _():
        wa.wait()

      @pl.when(jnp.logical_not(even))
      def _():
        wb.wait()

    for b in range(AHEAD):
      gather(b, b)

    def body(j, _):
      g = j + AHEAD
      buf_g = lax.rem(g, NBUF)

      @pl.when(g < NCHUNK)
      def _refill():
        @pl.when(j >= SLACK)
        def _drain():
          write_wait(j - SLACK, buf_g)

        gather(g, buf_g)

      buf = lax.rem(j, NBUF)
      gather_wait(j, buf)
      write(j, buf)
      return _

    lax.fori_loop(0, NCHUNK, body, None)

    for t in range(NBUF):
      j = NCHUNK - NBUF + t
      write_wait(j, j % NBUF)

  return k(table, x)


def kernel(x, table):
  return _sc_gather(x.astype(jnp.int32), table)


# speedup vs baseline: 1.3528x; 1.3247x over previous
import functools

import jax
import jax.numpy as jnp
from jax import lax
from jax.experimental import pallas as pl
from jax.experimental.pallas import tpu as pltpu
from jax.experimental.pallas import tpu_sc as plsc

VOCAB = 1000000
D = 64
R, C = 4096, 200
CA, CB = 128, C - 128
NC, NS = 2, 16
NW = NC * NS
RPW = R // NW
NCHUNK = RPW * 2
NBUF = 6
AHEAD = 4
SLACK = NBUF - AHEAD


def _sc_gather(xa, xb, table):
  mesh = plsc.VectorSubcoreMesh(
      core_axis_name="c", subcore_axis_name="s", num_cores=NC, num_subcores=NS
  )

  @functools.partial(
      pl.kernel,
      out_type=jax.ShapeDtypeStruct((R, C, 128), jnp.float32),
      mesh=mesh,
      scratch_types=[
          pltpu.VMEM((RPW, CA), jnp.int32),
          pltpu.VMEM((RPW, CB), jnp.int32),
          pltpu.VMEM((NBUF, CA, D), jnp.float32),
          pltpu.SemaphoreType.DMA((NBUF,)),
          pltpu.SemaphoreType.DMA((NBUF,)),
      ],
      compiler_params=pltpu.CompilerParams(use_tc_tiling_on_sc=False),
  )
  def k(table_hbm, xa_hbm, xb_hbm, out_hbm, xa_v, xb_v, rows_v, gsem, wsem):
    wid = lax.axis_index("s") * NC + lax.axis_index("c")
    row0 = wid * RPW
    pltpu.sync_copy(xa_hbm.at[pl.ds(row0, RPW)], xa_v)
    pltpu.sync_copy(xb_hbm.at[pl.ds(row0, RPW)], xb_v)

    def g_desc(r, buf, even):
      if even:
        return pltpu.make_async_copy(
            table_hbm.at[xa_v.at[r]], rows_v.at[buf], gsem.at[buf])
      return pltpu.make_async_copy(
          table_hbm.at[xb_v.at[r]],
          rows_v.at[buf, pl.ds(0, CB)],
          gsem.at[buf])

    def w_desc(r, buf, even):
      if even:
        return pltpu.make_async_copy(
            rows_v.at[buf],
            out_hbm.at[row0 + r, pl.ds(0, CA), pl.ds(0, D)],
            wsem.at[buf])
      return pltpu.make_async_copy(
          rows_v.at[buf, pl.ds(0, CB)],
          out_hbm.at[row0 + r, pl.ds(CA, CB), pl.ds(0, D)],
          wsem.at[buf])

    def by_parity(j, buf, mk, act):
      r = lax.div(j, 2)
      even = lax.rem(j, 2) == 0

      @pl.when(even)
      def _():
        act(mk(r, buf, True))

      @pl.when(jnp.logical_not(even))
      def _():
        act(mk(r, buf, False))

    gather = lambda j, buf: by_parity(j, buf, g_desc, lambda d: d.start())
    gather_wait = lambda j, buf: by_parity(j, buf, g_desc, lambda d: d.wait())
    write = lambda j, buf: by_parity(j, buf, w_desc, lambda d: d.start())
    write_wait = lambda j, buf: by_parity(j, buf, w_desc, lambda d: d.wait())

    for b in range(AHEAD):
      gather(b, b)

    def body(j, _):
      g = j + AHEAD
      buf_g = lax.rem(g, NBUF)

      @pl.when(g < NCHUNK)
      def _refill():
        @pl.when(j >= SLACK)
        def _drain():
          write_wait(j - SLACK, buf_g)

        gather(g, buf_g)

      buf = lax.rem(j, NBUF)
      gather_wait(j, buf)
      write(j, buf)
      return _

    lax.fori_loop(0, NCHUNK, body, None)

    for t in range(NBUF):
      j = NCHUNK - NBUF + t
      write_wait(j, j % NBUF)

  return k(table, xa, xb)


def kernel(x, table):
  x = x.astype(jnp.int32)
  out128 = _sc_gather(x[:, :CA], x[:, CA:], table)
  return out128[..., :D]
